# baseline (speedup 1.0000x reference)
import jax
import jax.numpy as jnp
from jax.experimental import pallas as pl
from jax.experimental.pallas import tpu as pltpu

_CHUNK = 1024
_DEPTH = 5
_N_CHUNKS = 16
_TBL_ROWS = 4096


def _body(x_hbm, t_hbm, o_hbm, in_buf, tbl, out_buf, in_sems, out_sems, tbl_sems):
    def tbl_copy(p):
        return pltpu.make_async_copy(
            t_hbm.at[pl.ds(p * _CHUNK, _CHUNK), :],
            tbl.at[pl.ds(p * _CHUNK, _CHUNK), :],
            tbl_sems.at[p],
        )

    tbl_copy(0).start()

    def in_copy(c):
        slot = c % _DEPTH
        return pltpu.make_async_copy(
            x_hbm.at[pl.ds(c * _CHUNK, _CHUNK), :],
            in_buf.at[slot],
            in_sems.at[slot],
        )

    def out_copy(c):
        slot = c % _DEPTH
        return pltpu.make_async_copy(
            out_buf.at[slot],
            o_hbm.at[pl.ds(c * _CHUNK, _CHUNK), :],
            out_sems.at[slot],
        )

    for c in range(_DEPTH):
        in_copy(c).start()
    for p in range(1, _TBL_ROWS // _CHUNK):
        tbl_copy(p).start()

    for c in range(_N_CHUNKS):
        slot = c % _DEPTH
        in_copy(c).wait()
        if c < _TBL_ROWS // _CHUNK:
            tbl_copy(c).wait()
        if c >= _DEPTH:
            out_copy(c - _DEPTH).wait()
        off = (c % (_TBL_ROWS // _CHUNK)) * _CHUNK
        out_buf[slot] = in_buf[slot] + tbl[pl.ds(off, _CHUNK), :]
        out_copy(c).start()
        nxt = c + _DEPTH
        if nxt < _N_CHUNKS:
            in_copy(nxt).start()

    for c in range(_N_CHUNKS - _DEPTH, _N_CHUNKS):
        out_copy(c).wait()


def kernel(inputs, pos_table):
    batch, seq_len, out_dim = inputs.shape
    flat = inputs.reshape(batch * seq_len, out_dim)
    out = pl.pallas_call(
        _body,
        in_specs=[
            pl.BlockSpec(memory_space=pltpu.MemorySpace.HBM),
            pl.BlockSpec(memory_space=pltpu.MemorySpace.HBM),
        ],
        out_specs=pl.BlockSpec(memory_space=pltpu.MemorySpace.HBM),
        out_shape=jax.ShapeDtypeStruct(flat.shape, flat.dtype),
        scratch_shapes=[
            pltpu.VMEM((_DEPTH, _CHUNK, out_dim), jnp.float32),
            pltpu.VMEM((seq_len, out_dim), jnp.float32),
            pltpu.VMEM((_DEPTH, _CHUNK, out_dim), jnp.float32),
            pltpu.SemaphoreType.DMA((_DEPTH,)),
            pltpu.SemaphoreType.DMA((_DEPTH,)),
            pltpu.SemaphoreType.DMA((_TBL_ROWS // _CHUNK,)),
        ],
    )(flat, pos_table)
    return out.reshape(batch, seq_len, out_dim)
